# baseline (speedup 1.0000x reference)
import jax
import jax.numpy as jnp
import numpy as np
from jax import lax
from jax.experimental import pallas as pl
from jax.experimental.pallas import tpu as pltpu
from jax.experimental.pallas import tpu_sc as plsc

VMIN = np.float32(-100000.0)
VMAX = np.float32(100000.0)
RANGE = np.float32(200000.0)
NBUCKETS = 100000000
DIV = np.float32(NBUCKETS - 1)
WIN = 64

ROWS = 16384
COLS = 512
BLOCK_ROWS = 4096

_NC = 2
_NS = 16
_NL = 16
_NW = _NC * _NS
_QPW = ROWS // _NW
_VPW = _QPW // _NL


def _bucket_vals(idx):
    t = idx.astype(jnp.float32) / DIV
    b = VMIN * (np.float32(1.0) - t) + VMAX * t
    return jnp.where(idx == NBUCKETS - 1, VMAX, b)


def _seeds_body(z_hbm, out_hbm, q_v, s_v):
    wid = lax.axis_index("s") * _NC + lax.axis_index("c")
    base0 = wid * _QPW
    pltpu.sync_copy(z_hbm.at[pl.ds(base0, _QPW)], q_v)

    def body(v, carry):
        q = q_v[pl.ds(v * _NL, _NL)]
        g = (q - VMIN) / RANGE * DIV
        base = jnp.clip(g.astype(jnp.int32) - WIN // 2, 0, NBUCKETS - WIN)
        res = jnp.zeros((_NL,), jnp.int32)
        w = WIN // 2
        while w >= 1:
            b = _bucket_vals(base + (res + (w - 1)))
            res = jnp.where(b < q, res + w, res)
            w //= 2
        b = _bucket_vals(base + res)
        res = jnp.where(b < q, res + 1, res)
        s_v[pl.ds(v * _NL, _NL)] = base + res
        return carry

    lax.fori_loop(0, _VPW, body, 0)
    pltpu.sync_copy(s_v, out_hbm.at[pl.ds(base0, _QPW)])


_seeds_call = pl.kernel(
    _seeds_body,
    mesh=plsc.VectorSubcoreMesh(core_axis_name="c", subcore_axis_name="s"),
    out_type=jax.ShapeDtypeStruct((ROWS,), jnp.int32),
    scratch_types=[
        pltpu.VMEM((_QPW,), jnp.float32),
        pltpu.VMEM((_QPW,), jnp.int32),
    ],
)


def _tile_body(s_ref, out_ref):
    out_ref[:, :] = jnp.broadcast_to(s_ref[:, :], (BLOCK_ROWS, COLS))


def kernel(z, c):
    del c
    seeds = _seeds_call(z[:, 0])
    return pl.pallas_call(
        _tile_body,
        grid=(ROWS // BLOCK_ROWS,),
        in_specs=[pl.BlockSpec((BLOCK_ROWS, 1), lambda i: (i, 0))],
        out_specs=pl.BlockSpec((BLOCK_ROWS, COLS), lambda i: (i, 0)),
        out_shape=jax.ShapeDtypeStruct((ROWS, COLS), jnp.int32),
    )(seeds[:, None])
